# baseline (speedup 1.0000x reference)
import functools

import jax
import jax.numpy as jnp
from jax import lax
from jax.experimental import pallas as pl
from jax.experimental.pallas import tpu as pltpu
from jax.experimental.pallas import tpu_sc as plsc

B = 2
C = 8
T = 8
N = 512
M = 3 * N
K = 16
QK = 32
VD = 32
NL = 2

SC_CORES = 2
SC_SUBCORES = 16
SC_WORKERS = SC_CORES * SC_SUBCORES
TOTAL_ROWS = B * T * K * N
ROWS_PER_W = TOTAL_ROWS // SC_WORKERS
IDX_CHUNK = 128
CHUNKS_PER_W = ROWS_PER_W // IDX_CHUNK


def _topk_body(inp_ref, prevT_ref, curT_ref, nextT_ref, gidx_ref):
    b = pl.program_id(0)
    t = pl.program_id(1)
    selfc = inp_ref[0, 0]
    poolT = jnp.concatenate(
        [prevT_ref[0, 0], curT_ref[0, 0], nextT_ref[0, 0]], axis=0
    )

    d2 = jnp.zeros((M, N), jnp.float32)
    for c in range(3):
        diff = selfc[c:c + 1, :] - poolT[:, c:c + 1]
        d2 = d2 + diff * diff

    iota = lax.broadcasted_iota(jnp.int32, (M, N), 0)
    iota_n = lax.broadcasted_iota(jnp.int32, (1, N), 1)
    idx0 = jnp.where(t == 0, iota_n, iota_n + N)
    d2 = jnp.where(iota == idx0, jnp.inf, d2)
    rows = [idx0]
    for _ in range(K - 1):
        mval = jnp.min(d2, axis=0, keepdims=True)
        eq = d2 == mval
        idx_j = jnp.min(jnp.where(eq, iota, M), axis=0, keepdims=True)
        rows.append(idx_j)
        d2 = jnp.where(iota == idx_j, jnp.inf, d2)
    idx = jnp.concatenate(rows, axis=0)

    s = idx >> 9
    nn = idx & (N - 1)
    tt = jnp.clip(t - 1 + s, 0, T - 1)
    grow = ((b * T + tt) << 9) | nn
    gidx_ref[0] = grow


def _run_topk(inpR, inpT, interpret=False):
    return pl.pallas_call(
        _topk_body,
        grid=(B, T),
        in_specs=[
            pl.BlockSpec((1, 1, C, N), lambda b, t: (b, t, 0, 0)),
            pl.BlockSpec((1, 1, N, C),
                         lambda b, t: (b, jnp.maximum(t - 1, 0), 0, 0)),
            pl.BlockSpec((1, 1, N, C), lambda b, t: (b, t, 0, 0)),
            pl.BlockSpec((1, 1, N, C),
                         lambda b, t: (b, jnp.minimum(t + 1, T - 1), 0, 0)),
        ],
        out_specs=pl.BlockSpec((1, K, N), lambda b, t: (b * T + t, 0, 0)),
        out_shape=jax.ShapeDtypeStruct((B * T, K, N), jnp.int32),
        interpret=interpret,
    )(inpR, inpT, inpT, inpT)


def _sc_gather_body(table_hbm, idx_hbm, out_hbm, idx_v, rows_v, sem):
    wid = lax.axis_index("s") * SC_CORES + lax.axis_index("c")
    base = wid * ROWS_PER_W
    pltpu.sync_copy(idx_hbm.at[pl.ds(base, ROWS_PER_W)], idx_v)
    pltpu.async_copy(table_hbm.at[idx_v], rows_v, sem).wait()
    pltpu.sync_copy(rows_v, out_hbm.at[pl.ds(base, ROWS_PER_W)])


def _run_sc_gather(table, gidx2d):
    mesh = plsc.VectorSubcoreMesh(core_axis_name="c", subcore_axis_name="s")
    f = functools.partial(
        pl.kernel,
        out_type=jax.ShapeDtypeStruct((TOTAL_ROWS, C), jnp.float32),
        mesh=mesh,
        scratch_types=[
            pltpu.VMEM((ROWS_PER_W,), jnp.int32),
            pltpu.VMEM((ROWS_PER_W, C), jnp.float32),
            pltpu.SemaphoreType.DMA,
        ],
        compiler_params=pltpu.CompilerParams(use_tc_tiling_on_sc=False),
    )(_sc_gather_body)
    return f(table, gidx2d)


def _attn_body(inpR_ref, gsel_ref, WqkK_ref, bqkK_ref, WvK_ref, bvK_ref,
               out0_ref, out1_ref):
    selfc = inpR_ref[0, 0]
    g = gsel_ref[0]
    selfrep = jnp.concatenate(
        [jnp.broadcast_to(selfc[c:c + 1, :], (K, N)) for c in range(C)],
        axis=0)
    x_r = jnp.concatenate(
        [selfrep[:4 * K] - g[:4 * K], selfrep[4 * K:], g[4 * K:]],
        axis=0)

    for l, out_ref in ((0, out0_ref), (1, out1_ref)):
        comb = (jnp.dot(WqkK_ref[l], x_r, preferred_element_type=jnp.float32)
                + bqkK_ref[:, l:l + 1])
        vv = (jnp.dot(WvK_ref[l], x_r, preferred_element_type=jnp.float32)
              + bvK_ref[:, l:l + 1])

        e = (comb[0:K][:, None, :] * comb[QK * K:QK * K + K][None, :, :])
        for c in range(1, QK):
            e = e + (comb[c * K:(c + 1) * K][:, None, :]
                     * comb[(QK + c) * K:(QK + c + 1) * K][None, :, :])
        mx = jnp.max(e, axis=1, keepdims=True)
        p = jnp.exp(e - mx)
        den = jnp.sum(p, axis=1)
        w = p[:, 0, :] / den

        wt = jnp.concatenate([w] * VD, axis=0)
        wv = (vv * wt).reshape(VD, K, N)
        out = jnp.sum(wv, axis=1)
        out_ref[0, 0] = jnp.concatenate([selfc[0:4], out], axis=0)


def _run_attn(inpR, gsel3, WqkK, bqkK, WvK, bvK, interpret=False):
    return pl.pallas_call(
        _attn_body,
        grid=(B, T),
        in_specs=[
            pl.BlockSpec((1, 1, C, N), lambda b, t: (b, t, 0, 0)),
            pl.BlockSpec((1, C * K, N), lambda b, t: (b * T + t, 0, 0)),
            pl.BlockSpec((NL, 2 * QK * K, 12 * K), lambda b, t: (0, 0, 0)),
            pl.BlockSpec((2 * QK * K, NL), lambda b, t: (0, 0)),
            pl.BlockSpec((NL, VD * K, 12 * K), lambda b, t: (0, 0, 0)),
            pl.BlockSpec((VD * K, NL), lambda b, t: (0, 0)),
        ],
        out_specs=[
            pl.BlockSpec((1, 1, 4 + VD, N), lambda b, t: (b, t, 0, 0)),
            pl.BlockSpec((1, 1, 4 + VD, N), lambda b, t: (b, t, 0, 0)),
        ],
        out_shape=[
            jax.ShapeDtypeStruct((B, T, 4 + VD, N), jnp.float32),
            jax.ShapeDtypeStruct((B, T, 4 + VD, N), jnp.float32),
        ],
        interpret=interpret,
    )(inpR, gsel3, WqkK, bqkK, WvK, bvK)


def _expand_weights(Wqk, bqk, Wv, bv):
    eye = jnp.eye(K, dtype=jnp.float32)
    alpha = QK ** -0.25
    WqkK = (jnp.einsum('loc,kj->lokcj', Wqk, eye)
            .reshape(NL, 2 * QK * K, 12 * K) * alpha)
    WvK = jnp.einsum('loc,kj->lokcj', Wv, eye).reshape(NL, VD * K, 12 * K)
    bqkK = jnp.repeat(bqk, K, axis=1).T * alpha
    bvK = jnp.repeat(bv, K, axis=1).T
    return WqkK, bqkK, WvK, bvK


def kernel(input_tensor, Wqk, bqk, Wv, bv):
    inpT = jnp.transpose(input_tensor, (0, 2, 3, 1))
    inpR = jnp.transpose(input_tensor, (0, 2, 1, 3))

    gidx = _run_topk(inpR, inpT)
    rows = _run_sc_gather(inpT.reshape(B * T * N, C),
                          gidx.reshape(TOTAL_ROWS))

    gsel3 = jnp.transpose(rows.reshape(B * T, K, N, C),
                          (0, 3, 1, 2)).reshape(B * T, C * K, N)

    WqkK, bqkK, WvK, bvK = _expand_weights(Wqk, bqk, Wv, bv)
    o0, o1 = _run_attn(inpR, gsel3, WqkK, bqkK, WvK, bvK)
    return (jnp.transpose(o0, (0, 2, 1, 3)), jnp.transpose(o1, (0, 2, 1, 3)))


# speedup vs baseline: 1.2023x; 1.2023x over previous
import functools

import jax
import jax.numpy as jnp
from jax import lax
from jax.experimental import pallas as pl
from jax.experimental.pallas import tpu as pltpu
from jax.experimental.pallas import tpu_sc as plsc

B = 2
C = 8
T = 8
N = 512
M = 3 * N
K = 16
QK = 32
VD = 32
NL = 2

SC_CORES = 2
SC_SUBCORES = 16
SC_WORKERS = SC_CORES * SC_SUBCORES
TOTAL_ROWS = B * T * K * N
ROWS_PER_W = TOTAL_ROWS // SC_WORKERS
IDX_CHUNK = 128
CHUNKS_PER_W = ROWS_PER_W // IDX_CHUNK


def _topk_body(inp_ref, prevT_ref, curT_ref, nextT_ref, gidx_ref):
    b = pl.program_id(0)
    t = pl.program_id(1)
    selfc = inp_ref[0, 0]
    poolT = jnp.concatenate(
        [prevT_ref[0, 0], curT_ref[0, 0], nextT_ref[0, 0]], axis=0
    )

    d2 = jnp.zeros((M, N), jnp.float32)
    for c in range(3):
        diff = selfc[c:c + 1, :] - poolT[:, c:c + 1]
        d2 = d2 + diff * diff

    iota = lax.broadcasted_iota(jnp.int32, (M, N), 0)
    iota_n = lax.broadcasted_iota(jnp.int32, (1, N), 1)
    idx0 = jnp.where(t == 0, iota_n, iota_n + N)
    d2 = jnp.where(iota == idx0, jnp.inf, d2)
    rows = [idx0]
    for _ in range(K - 1):
        mval = jnp.min(d2, axis=0, keepdims=True)
        eq = d2 == mval
        idx_j = jnp.min(jnp.where(eq, iota, M), axis=0, keepdims=True)
        rows.append(idx_j)
        d2 = jnp.where(iota == idx_j, jnp.inf, d2)
    idx = jnp.concatenate(rows, axis=0)

    s = idx >> 9
    nn = idx & (N - 1)
    tt = jnp.clip(t - 1 + s, 0, T - 1)
    grow = ((b * T + tt) << 9) | nn
    gidx_ref[0] = grow


def _run_topk(inpR, inpT, interpret=False):
    return pl.pallas_call(
        _topk_body,
        grid=(B, T),
        in_specs=[
            pl.BlockSpec((1, 1, C, N), lambda b, t: (b, t, 0, 0)),
            pl.BlockSpec((1, 1, N, C),
                         lambda b, t: (b, jnp.maximum(t - 1, 0), 0, 0)),
            pl.BlockSpec((1, 1, N, C), lambda b, t: (b, t, 0, 0)),
            pl.BlockSpec((1, 1, N, C),
                         lambda b, t: (b, jnp.minimum(t + 1, T - 1), 0, 0)),
        ],
        out_specs=pl.BlockSpec((1, K, N), lambda b, t: (b * T + t, 0, 0)),
        out_shape=jax.ShapeDtypeStruct((B * T, K, N), jnp.int32),
        interpret=interpret,
    )(inpR, inpT, inpT, inpT)


def _sc_gather_body(table_hbm, idx_hbm, out_hbm, win_v, idx_v, rows_v):
    wid = lax.axis_index("s") * SC_CORES + lax.axis_index("c")
    bt = wid // 2
    b = bt // T
    t = bt % T
    wbt = jnp.clip(t - 1, 0, T - 3)
    wb = (b * T + wbt) * N
    pltpu.sync_copy(table_hbm.at[pl.ds(wb * C, M * C)], win_v)
    pltpu.sync_copy(idx_hbm.at[pl.ds(wid * ROWS_PER_W, ROWS_PER_W)], idx_v)

    def step(i, carry):
        locb = (idx_v[pl.ds(i * 16, 16)] - wb) * C
        for c in range(C):
            rows_v[c, pl.ds(i * 16, 16)] = plsc.load_gather(win_v, [locb + c])
        return carry

    lax.fori_loop(0, ROWS_PER_W // 16, step, 0)
    pltpu.sync_copy(rows_v, out_hbm.at[pl.ds(wid * C, C)])


def _run_sc_gather(table, gidx_flat):
    mesh = plsc.VectorSubcoreMesh(core_axis_name="c", subcore_axis_name="s")
    f = functools.partial(
        pl.kernel,
        out_type=jax.ShapeDtypeStruct((SC_WORKERS * C, ROWS_PER_W), jnp.float32),
        mesh=mesh,
        scratch_types=[
            pltpu.VMEM((M * C,), jnp.float32),
            pltpu.VMEM((ROWS_PER_W,), jnp.int32),
            pltpu.VMEM((C, ROWS_PER_W), jnp.float32),
        ],
        compiler_params=pltpu.CompilerParams(needs_layout_passes=False),
    )(_sc_gather_body)
    return f(table, gidx_flat)


def _attn_body(inpR_ref, gsel_ref, WqkK_ref, bqkK_ref, WvK_ref, bvK_ref,
               out0_ref, out1_ref):
    selfc = inpR_ref[0, 0]
    g = gsel_ref[0]
    selfrep = jnp.concatenate(
        [jnp.broadcast_to(selfc[c:c + 1, :], (K, N)) for c in range(C)],
        axis=0)
    x_r = jnp.concatenate(
        [selfrep[:4 * K] - g[:4 * K], selfrep[4 * K:], g[4 * K:]],
        axis=0)

    for l, out_ref in ((0, out0_ref), (1, out1_ref)):
        comb = (jnp.dot(WqkK_ref[l], x_r, preferred_element_type=jnp.float32)
                + bqkK_ref[:, l:l + 1])
        vv = (jnp.dot(WvK_ref[l], x_r, preferred_element_type=jnp.float32)
              + bvK_ref[:, l:l + 1])

        e = (comb[0:K][:, None, :] * comb[QK * K:QK * K + K][None, :, :])
        for c in range(1, QK):
            e = e + (comb[c * K:(c + 1) * K][:, None, :]
                     * comb[(QK + c) * K:(QK + c + 1) * K][None, :, :])
        mx = jnp.max(e, axis=1, keepdims=True)
        p = jnp.exp(e - mx)
        den = jnp.sum(p, axis=1)
        w = p[:, 0, :] / den

        wt = jnp.concatenate([w] * VD, axis=0)
        wv = (vv * wt).reshape(VD, K, N)
        out = jnp.sum(wv, axis=1)
        out_ref[0, 0] = jnp.concatenate([selfc[0:4], out], axis=0)


def _run_attn(inpR, gsel3, WqkK, bqkK, WvK, bvK, interpret=False):
    return pl.pallas_call(
        _attn_body,
        grid=(B, T),
        in_specs=[
            pl.BlockSpec((1, 1, C, N), lambda b, t: (b, t, 0, 0)),
            pl.BlockSpec((1, C * K, N), lambda b, t: (b * T + t, 0, 0)),
            pl.BlockSpec((NL, 2 * QK * K, 12 * K), lambda b, t: (0, 0, 0)),
            pl.BlockSpec((2 * QK * K, NL), lambda b, t: (0, 0)),
            pl.BlockSpec((NL, VD * K, 12 * K), lambda b, t: (0, 0, 0)),
            pl.BlockSpec((VD * K, NL), lambda b, t: (0, 0)),
        ],
        out_specs=[
            pl.BlockSpec((1, 1, 4 + VD, N), lambda b, t: (b, t, 0, 0)),
            pl.BlockSpec((1, 1, 4 + VD, N), lambda b, t: (b, t, 0, 0)),
        ],
        out_shape=[
            jax.ShapeDtypeStruct((B, T, 4 + VD, N), jnp.float32),
            jax.ShapeDtypeStruct((B, T, 4 + VD, N), jnp.float32),
        ],
        interpret=interpret,
    )(inpR, gsel3, WqkK, bqkK, WvK, bvK)


def _expand_weights(Wqk, bqk, Wv, bv):
    eye = jnp.eye(K, dtype=jnp.float32)
    alpha = QK ** -0.25
    WqkK = (jnp.einsum('loc,kj->lokcj', Wqk, eye)
            .reshape(NL, 2 * QK * K, 12 * K) * alpha)
    WvK = jnp.einsum('loc,kj->lokcj', Wv, eye).reshape(NL, VD * K, 12 * K)
    bqkK = jnp.repeat(bqk, K, axis=1).T * alpha
    bvK = jnp.repeat(bv, K, axis=1).T
    return WqkK, bqkK, WvK, bvK


def kernel(input_tensor, Wqk, bqk, Wv, bv):
    inpT = jnp.transpose(input_tensor, (0, 2, 3, 1))
    inpR = jnp.transpose(input_tensor, (0, 2, 1, 3))

    gidx = _run_topk(inpR, inpT)
    rows = _run_sc_gather(inpT.reshape(B * T * N * C),
                          gidx.reshape(TOTAL_ROWS))

    gsel3 = jnp.transpose(rows.reshape(B * T, 2, C, K // 2, N),
                          (0, 2, 1, 3, 4)).reshape(B * T, C * K, N)

    WqkK, bqkK, WvK, bvK = _expand_weights(Wqk, bqk, Wv, bv)
    o0, o1 = _run_attn(inpR, gsel3, WqkK, bqkK, WvK, bvK)
    return (jnp.transpose(o0, (0, 2, 1, 3)), jnp.transpose(o1, (0, 2, 1, 3)))


# speedup vs baseline: 1.2056x; 1.0027x over previous
import functools

import jax
import jax.numpy as jnp
from jax import lax
from jax.experimental import pallas as pl
from jax.experimental.pallas import tpu as pltpu
from jax.experimental.pallas import tpu_sc as plsc

B = 2
C = 8
T = 8
N = 512
M = 3 * N
K = 16
QK = 32
VD = 32
NL = 2

SC_CORES = 2
SC_SUBCORES = 16
SC_WORKERS = SC_CORES * SC_SUBCORES
TOTAL_ROWS = B * T * K * N
ROWS_PER_W = TOTAL_ROWS // SC_WORKERS
IDX_CHUNK = 128
CHUNKS_PER_W = ROWS_PER_W // IDX_CHUNK


def _topk_body(inp_ref, prevT_ref, curT_ref, nextT_ref, gidx_ref):
    b = pl.program_id(0)
    t = pl.program_id(1)
    selfc = inp_ref[0, 0]
    poolT = jnp.concatenate(
        [prevT_ref[0, 0], curT_ref[0, 0], nextT_ref[0, 0]], axis=0
    )

    d2 = jnp.zeros((M, N), jnp.float32)
    for c in range(3):
        diff = selfc[c:c + 1, :] - poolT[:, c:c + 1]
        d2 = d2 + diff * diff

    iota = lax.broadcasted_iota(jnp.int32, (M, N), 0)
    iota_n = lax.broadcasted_iota(jnp.int32, (1, N), 1)
    idx0 = jnp.where(t == 0, iota_n, iota_n + N)
    d2 = jnp.where(iota == idx0, jnp.inf, d2)
    rows = [idx0]
    for _ in range(K - 1):
        mval = jnp.min(d2, axis=0, keepdims=True)
        eq = d2 == mval
        idx_j = jnp.min(jnp.where(eq, iota, M), axis=0, keepdims=True)
        rows.append(idx_j)
        d2 = jnp.where(iota == idx_j, jnp.inf, d2)
    idx = jnp.concatenate(rows, axis=0)

    s = idx >> 9
    nn = idx & (N - 1)
    tt = jnp.clip(t - 1 + s, 0, T - 1)
    grow = ((b * T + tt) << 9) | nn
    gidx_ref[0] = grow


def _run_topk(inpR, inpT, interpret=False):
    return pl.pallas_call(
        _topk_body,
        grid=(B, T),
        in_specs=[
            pl.BlockSpec((1, 1, C, N), lambda b, t: (b, t, 0, 0)),
            pl.BlockSpec((1, 1, N, C),
                         lambda b, t: (b, jnp.maximum(t - 1, 0), 0, 0)),
            pl.BlockSpec((1, 1, N, C), lambda b, t: (b, t, 0, 0)),
            pl.BlockSpec((1, 1, N, C),
                         lambda b, t: (b, jnp.minimum(t + 1, T - 1), 0, 0)),
        ],
        out_specs=pl.BlockSpec((1, K, N), lambda b, t: (b * T + t, 0, 0)),
        out_shape=jax.ShapeDtypeStruct((B * T, K, N), jnp.int32),
        interpret=interpret,
    )(inpR, inpT, inpT, inpT)


def _sc_gather_body(table_hbm, idx_hbm, out_hbm, win_v, idx_v, rows_v):
    wid = lax.axis_index("s") * SC_CORES + lax.axis_index("c")
    bt = wid // 2
    b = bt // T
    t = bt % T
    wbt = jnp.clip(t - 1, 0, T - 3)
    wb = (b * T + wbt) * N
    pltpu.sync_copy(table_hbm.at[pl.ds(wb * C, M * C)], win_v)
    pltpu.sync_copy(idx_hbm.at[pl.ds(wid * ROWS_PER_W, ROWS_PER_W)], idx_v)

    @plsc.parallel_loop(0, ROWS_PER_W // 16, unroll=8)
    def _gather_step(i):
        locb = (idx_v[pl.ds(i * 16, 16)] - wb) * C
        for c in range(C):
            rows_v[c, pl.ds(i * 16, 16)] = plsc.load_gather(win_v, [locb + c])
    pltpu.sync_copy(rows_v, out_hbm.at[pl.ds(wid * C, C)])


def _run_sc_gather(table, gidx_flat):
    mesh = plsc.VectorSubcoreMesh(core_axis_name="c", subcore_axis_name="s")
    f = functools.partial(
        pl.kernel,
        out_type=jax.ShapeDtypeStruct((SC_WORKERS * C, ROWS_PER_W), jnp.float32),
        mesh=mesh,
        scratch_types=[
            pltpu.VMEM((M * C,), jnp.float32),
            pltpu.VMEM((ROWS_PER_W,), jnp.int32),
            pltpu.VMEM((C, ROWS_PER_W), jnp.float32),
        ],
        compiler_params=pltpu.CompilerParams(needs_layout_passes=False),
    )(_sc_gather_body)
    return f(table, gidx_flat)


def _attn_body(inpR_ref, gsel_ref, WqkK_ref, bqkK_ref, WvK_ref, bvK_ref,
               out0_ref, out1_ref):
    selfc = inpR_ref[0, 0]
    g = gsel_ref[0]
    selfrep = jnp.concatenate(
        [jnp.broadcast_to(selfc[c:c + 1, :], (K, N)) for c in range(C)],
        axis=0)
    x_r = jnp.concatenate(
        [selfrep[:4 * K] - g[:4 * K], selfrep[4 * K:], g[4 * K:]],
        axis=0)

    for l, out_ref in ((0, out0_ref), (1, out1_ref)):
        comb = (jnp.dot(WqkK_ref[l], x_r, preferred_element_type=jnp.float32)
                + bqkK_ref[:, l:l + 1])
        vv = (jnp.dot(WvK_ref[l], x_r, preferred_element_type=jnp.float32)
              + bvK_ref[:, l:l + 1])

        e = (comb[0:K][:, None, :] * comb[QK * K:QK * K + K][None, :, :])
        for c in range(1, QK):
            e = e + (comb[c * K:(c + 1) * K][:, None, :]
                     * comb[(QK + c) * K:(QK + c + 1) * K][None, :, :])
        mx = jnp.max(e, axis=1, keepdims=True)
        p = jnp.exp(e - mx)
        den = jnp.sum(p, axis=1)
        w = p[:, 0, :] / den

        wt = jnp.concatenate([w] * VD, axis=0)
        wv = (vv * wt).reshape(VD, K, N)
        out = jnp.sum(wv, axis=1)
        out_ref[0, 0] = jnp.concatenate([selfc[0:4], out], axis=0)


def _run_attn(inpR, gsel3, WqkK, bqkK, WvK, bvK, interpret=False):
    return pl.pallas_call(
        _attn_body,
        grid=(B, T),
        in_specs=[
            pl.BlockSpec((1, 1, C, N), lambda b, t: (b, t, 0, 0)),
            pl.BlockSpec((1, C * K, N), lambda b, t: (b * T + t, 0, 0)),
            pl.BlockSpec((NL, 2 * QK * K, 12 * K), lambda b, t: (0, 0, 0)),
            pl.BlockSpec((2 * QK * K, NL), lambda b, t: (0, 0)),
            pl.BlockSpec((NL, VD * K, 12 * K), lambda b, t: (0, 0, 0)),
            pl.BlockSpec((VD * K, NL), lambda b, t: (0, 0)),
        ],
        out_specs=[
            pl.BlockSpec((1, 1, 4 + VD, N), lambda b, t: (b, t, 0, 0)),
            pl.BlockSpec((1, 1, 4 + VD, N), lambda b, t: (b, t, 0, 0)),
        ],
        out_shape=[
            jax.ShapeDtypeStruct((B, T, 4 + VD, N), jnp.float32),
            jax.ShapeDtypeStruct((B, T, 4 + VD, N), jnp.float32),
        ],
        interpret=interpret,
    )(inpR, gsel3, WqkK, bqkK, WvK, bvK)


def _expand_weights(Wqk, bqk, Wv, bv):
    eye = jnp.eye(K, dtype=jnp.float32)
    alpha = QK ** -0.25
    WqkK = (jnp.einsum('loc,kj->lokcj', Wqk, eye)
            .reshape(NL, 2 * QK * K, 12 * K) * alpha)
    WvK = jnp.einsum('loc,kj->lokcj', Wv, eye).reshape(NL, VD * K, 12 * K)
    bqkK = jnp.repeat(bqk, K, axis=1).T * alpha
    bvK = jnp.repeat(bv, K, axis=1).T
    return WqkK, bqkK, WvK, bvK


def kernel(input_tensor, Wqk, bqk, Wv, bv):
    inpT = jnp.transpose(input_tensor, (0, 2, 3, 1))
    inpR = jnp.transpose(input_tensor, (0, 2, 1, 3))

    gidx = _run_topk(inpR, inpT)
    rows = _run_sc_gather(inpT.reshape(B * T * N * C),
                          gidx.reshape(TOTAL_ROWS))

    gsel3 = jnp.transpose(rows.reshape(B * T, 2, C, K // 2, N),
                          (0, 2, 1, 3, 4)).reshape(B * T, C * K, N)

    WqkK, bqkK, WvK, bvK = _expand_weights(Wqk, bqk, Wv, bv)
    o0, o1 = _run_attn(inpR, gsel3, WqkK, bqkK, WvK, bvK)
    return (jnp.transpose(o0, (0, 2, 1, 3)), jnp.transpose(o1, (0, 2, 1, 3)))


# speedup vs baseline: 1.3468x; 1.1171x over previous
import functools

import jax
import jax.numpy as jnp
from jax import lax
from jax.experimental import pallas as pl
from jax.experimental.pallas import tpu as pltpu
from jax.experimental.pallas import tpu_sc as plsc

B = 2
C = 8
T = 8
N = 512
M = 3 * N
K = 16
QK = 32
VD = 32
NL = 2

SC_CORES = 2
SC_SUBCORES = 16
SC_WORKERS = SC_CORES * SC_SUBCORES
TOTAL_ROWS = B * T * K * N
ROWS_PER_W = TOTAL_ROWS // SC_WORKERS
IDX_CHUNK = 128
CHUNKS_PER_W = ROWS_PER_W // IDX_CHUNK


def _topk_body(inp_ref, prevT_ref, curT_ref, nextT_ref, gidx_ref):
    b = pl.program_id(0)
    t = pl.program_id(1)
    selfc = inp_ref[0, 0]
    poolT = jnp.concatenate(
        [prevT_ref[0, 0], curT_ref[0, 0], nextT_ref[0, 0]], axis=0
    )

    d2 = jnp.zeros((M, N), jnp.float32)
    for c in range(3):
        diff = selfc[c:c + 1, :] - poolT[:, c:c + 1]
        d2 = d2 + diff * diff

    iota = lax.broadcasted_iota(jnp.int32, (M, N), 0)
    iota_n = lax.broadcasted_iota(jnp.int32, (1, N), 1)
    idx0 = jnp.where(t == 0, iota_n, iota_n + N)
    d2 = jnp.where(iota == idx0, jnp.inf, d2)
    rows = [idx0]
    for _ in range(K - 1):
        idx_j = jnp.argmin(d2, axis=0, keepdims=True).astype(jnp.int32)
        rows.append(idx_j)
        d2 = jnp.where(iota == idx_j, jnp.inf, d2)
    idx = jnp.concatenate(rows, axis=0)

    s = idx >> 9
    nn = idx & (N - 1)
    tt = jnp.clip(t - 1 + s, 0, T - 1)
    grow = ((b * T + tt) << 9) | nn
    gidx_ref[0] = grow


def _run_topk(inpR, inpT, interpret=False):
    return pl.pallas_call(
        _topk_body,
        grid=(B, T),
        in_specs=[
            pl.BlockSpec((1, 1, C, N), lambda b, t: (b, t, 0, 0)),
            pl.BlockSpec((1, 1, N, C),
                         lambda b, t: (b, jnp.maximum(t - 1, 0), 0, 0)),
            pl.BlockSpec((1, 1, N, C), lambda b, t: (b, t, 0, 0)),
            pl.BlockSpec((1, 1, N, C),
                         lambda b, t: (b, jnp.minimum(t + 1, T - 1), 0, 0)),
        ],
        out_specs=pl.BlockSpec((1, K, N), lambda b, t: (b * T + t, 0, 0)),
        out_shape=jax.ShapeDtypeStruct((B * T, K, N), jnp.int32),
        interpret=interpret,
    )(inpR, inpT, inpT, inpT)


def _sc_gather_body(table_hbm, idx_hbm, out_hbm, win_v, idx_v, rows_v):
    wid = lax.axis_index("s") * SC_CORES + lax.axis_index("c")
    bt = wid // 2
    b = bt // T
    t = bt % T
    wbt = jnp.clip(t - 1, 0, T - 3)
    wb = (b * T + wbt) * N
    pltpu.sync_copy(table_hbm.at[pl.ds(wb * C, M * C)], win_v)
    pltpu.sync_copy(idx_hbm.at[pl.ds(wid * ROWS_PER_W, ROWS_PER_W)], idx_v)

    @plsc.parallel_loop(0, ROWS_PER_W // 16, unroll=8)
    def _gather_step(i):
        locb = (idx_v[pl.ds(i * 16, 16)] - wb) * C
        for c in range(C):
            rows_v[c, pl.ds(i * 16, 16)] = plsc.load_gather(win_v, [locb + c])
    pltpu.sync_copy(rows_v, out_hbm.at[pl.ds(wid * C, C)])


def _run_sc_gather(table, gidx_flat):
    mesh = plsc.VectorSubcoreMesh(core_axis_name="c", subcore_axis_name="s")
    f = functools.partial(
        pl.kernel,
        out_type=jax.ShapeDtypeStruct((SC_WORKERS * C, ROWS_PER_W), jnp.float32),
        mesh=mesh,
        scratch_types=[
            pltpu.VMEM((M * C,), jnp.float32),
            pltpu.VMEM((ROWS_PER_W,), jnp.int32),
            pltpu.VMEM((C, ROWS_PER_W), jnp.float32),
        ],
        compiler_params=pltpu.CompilerParams(needs_layout_passes=False),
    )(_sc_gather_body)
    return f(table, gidx_flat)


def _attn_body(inpR_ref, gsel_ref, WqkK_ref, bqkK_ref, WvK_ref, bvK_ref,
               out0_ref, out1_ref):
    selfc = inpR_ref[0, 0]
    g = gsel_ref[0]
    selfrep = jnp.concatenate(
        [jnp.broadcast_to(selfc[c:c + 1, :], (K, N)) for c in range(C)],
        axis=0)
    x_r = jnp.concatenate(
        [selfrep[:4 * K] - g[:4 * K], selfrep[4 * K:], g[4 * K:]],
        axis=0)

    for l, out_ref in ((0, out0_ref), (1, out1_ref)):
        comb = (jnp.dot(WqkK_ref[l], x_r, preferred_element_type=jnp.float32)
                + bqkK_ref[:, l:l + 1])
        vv = (jnp.dot(WvK_ref[l], x_r, preferred_element_type=jnp.float32)
              + bvK_ref[:, l:l + 1])

        e = (comb[0:K][:, None, :] * comb[QK * K:QK * K + K][None, :, :])
        for c in range(1, QK):
            e = e + (comb[c * K:(c + 1) * K][:, None, :]
                     * comb[(QK + c) * K:(QK + c + 1) * K][None, :, :])
        mx = jnp.max(e, axis=1, keepdims=True)
        p = jnp.exp(e - mx)
        den = jnp.sum(p, axis=1)
        w = p[:, 0, :] / den

        wt = jnp.concatenate([w] * VD, axis=0)
        wv = (vv * wt).reshape(VD, K, N)
        out = jnp.sum(wv, axis=1)
        out_ref[0, 0] = jnp.concatenate([selfc[0:4], out], axis=0)


def _run_attn(inpR, gsel3, WqkK, bqkK, WvK, bvK, interpret=False):
    return pl.pallas_call(
        _attn_body,
        grid=(B, T),
        in_specs=[
            pl.BlockSpec((1, 1, C, N), lambda b, t: (b, t, 0, 0)),
            pl.BlockSpec((1, C * K, N), lambda b, t: (b * T + t, 0, 0)),
            pl.BlockSpec((NL, 2 * QK * K, 12 * K), lambda b, t: (0, 0, 0)),
            pl.BlockSpec((2 * QK * K, NL), lambda b, t: (0, 0)),
            pl.BlockSpec((NL, VD * K, 12 * K), lambda b, t: (0, 0, 0)),
            pl.BlockSpec((VD * K, NL), lambda b, t: (0, 0)),
        ],
        out_specs=[
            pl.BlockSpec((1, 1, 4 + VD, N), lambda b, t: (b, t, 0, 0)),
            pl.BlockSpec((1, 1, 4 + VD, N), lambda b, t: (b, t, 0, 0)),
        ],
        out_shape=[
            jax.ShapeDtypeStruct((B, T, 4 + VD, N), jnp.float32),
            jax.ShapeDtypeStruct((B, T, 4 + VD, N), jnp.float32),
        ],
        interpret=interpret,
    )(inpR, gsel3, WqkK, bqkK, WvK, bvK)


def _expand_weights(Wqk, bqk, Wv, bv):
    eye = jnp.eye(K, dtype=jnp.float32)
    alpha = QK ** -0.25
    WqkK = (jnp.einsum('loc,kj->lokcj', Wqk, eye)
            .reshape(NL, 2 * QK * K, 12 * K) * alpha)
    WvK = jnp.einsum('loc,kj->lokcj', Wv, eye).reshape(NL, VD * K, 12 * K)
    bqkK = jnp.repeat(bqk, K, axis=1).T * alpha
    bvK = jnp.repeat(bv, K, axis=1).T
    return WqkK, bqkK, WvK, bvK


def kernel(input_tensor, Wqk, bqk, Wv, bv):
    inpT = jnp.transpose(input_tensor, (0, 2, 3, 1))
    inpR = jnp.transpose(input_tensor, (0, 2, 1, 3))

    gidx = _run_topk(inpR, inpT)
    rows = _run_sc_gather(inpT.reshape(B * T * N * C),
                          gidx.reshape(TOTAL_ROWS))

    gsel3 = jnp.transpose(rows.reshape(B * T, 2, C, K // 2, N),
                          (0, 2, 1, 3, 4)).reshape(B * T, C * K, N)

    WqkK, bqkK, WvK, bvK = _expand_weights(Wqk, bqk, Wv, bv)
    o0, o1 = _run_attn(inpR, gsel3, WqkK, bqkK, WvK, bvK)
    return (jnp.transpose(o0, (0, 2, 1, 3)), jnp.transpose(o1, (0, 2, 1, 3)))
